# Initial kernel scaffold; baseline (speedup 1.0000x reference)
import jax
import jax.numpy as jnp
from jax.experimental import pallas as pl


def kernel(x, w1, b1, w2, b2):
    raise NotImplementedError("write your pallas kernel here")


# speedup vs baseline: 1.0449x; 1.0449x over previous
import jax
import jax.numpy as jnp
from jax.experimental import pallas as pl
from jax.experimental.pallas import tpu as pltpu

_TM = 512


def _ffn_kernel(x_ref, w1_ref, b1_ref, w2_ref, b2_ref, o_ref):
    xb = x_ref[...].astype(jnp.bfloat16)
    h = jnp.dot(xb, w1_ref[...], preferred_element_type=jnp.float32)
    h = jnp.maximum(h + b1_ref[...], 0.0).astype(jnp.bfloat16)
    out = jnp.dot(h, w2_ref[...], preferred_element_type=jnp.float32)
    o_ref[...] = out + b2_ref[...]


def kernel(x, w1, b1, w2, b2):
    B, S, d_model = x.shape
    d_ff = w1.shape[1]
    M = B * S

    x2d = x.reshape(M, d_model)
    w1b = w1.astype(jnp.bfloat16)
    w2b = w2.astype(jnp.bfloat16)
    b1_2d = b1.reshape(1, d_ff)
    b2_2d = b2.reshape(1, d_model)

    grid_m = M // _TM
    out2d = pl.pallas_call(
        _ffn_kernel,
        out_shape=jax.ShapeDtypeStruct((M, d_model), jnp.float32),
        grid=(grid_m,),
        in_specs=[
            pl.BlockSpec((_TM, d_model), lambda i: (i, 0)),
            pl.BlockSpec((d_model, d_ff), lambda i: (0, 0)),
            pl.BlockSpec((1, d_ff), lambda i: (0, 0)),
            pl.BlockSpec((d_ff, d_model), lambda i: (0, 0)),
            pl.BlockSpec((1, d_model), lambda i: (0, 0)),
        ],
        out_specs=pl.BlockSpec((_TM, d_model), lambda i: (i, 0)),
        compiler_params=pltpu.CompilerParams(
            dimension_semantics=("parallel",),
            vmem_limit_bytes=62 * 1024 * 1024,
        ),
        cost_estimate=pl.CostEstimate(
            flops=4 * M * d_model * d_ff,
            transcendentals=0,
            bytes_accessed=(x2d.size * 4 + w1b.size * 2 + w2b.size * 2
                            + b1.size * 4 + b2.size * 4 + M * d_model * 4),
        ),
    )(x2d, w1b, b1_2d, w2b, b2_2d)

    return out2d.reshape(B, S, d_model)
